# baseline (speedup 1.0000x reference)
import functools

import jax
import jax.numpy as jnp
from jax import lax
from jax.experimental import pallas as pl
from jax.experimental.pallas import tpu as pltpu
from jax.experimental.pallas import tpu_sc as plsc

_NUM_CORES = 2
_NUM_SUBCORES = 16
_LANES = 16
_NW = _NUM_CORES * _NUM_SUBCORES


def _memset_body(out_ref):
    out_ref[...] = jnp.zeros_like(out_ref)


@functools.cache
def _make_sc_scatter(BN, C):
    per_w = BN // _NW
    n_chunks = per_w // _LANES
    n_rows = n_chunks // 8
    mesh = plsc.VectorSubcoreMesh(core_axis_name="c", subcore_axis_name="s")

    @functools.partial(
        pl.kernel,
        out_type=(),
        mesh=mesh,
        scratch_types=[
            pltpu.VMEM((per_w,), jnp.int32),
            pltpu.VMEM((n_rows, 128), jnp.int32),
            pltpu.VMEM((n_rows, 128), jnp.float32),
            pltpu.SemaphoreType.DMA,
        ],
    )
    def sc_scatter(lab_hbm, buf_hbm, lab_v, off_v, val_v, sem):
        wid = lax.axis_index("s") * _NUM_CORES + lax.axis_index("c")
        base = wid * per_w
        pltpu.sync_copy(lab_hbm.at[pl.ds(base, per_w)], lab_v)
        iota = lax.iota(jnp.int32, _LANES)
        for k in range(n_chunks):
            lab = lab_v[pl.ds(k * _LANES, _LANES)]
            valid = lab >= 0
            off = (base + k * _LANES) * C + iota * C + jnp.where(valid, lab, 0)
            val = jnp.where(valid, jnp.float32(1.0), jnp.float32(0.0))
            r, cb = k // 8, (k % 8) * _LANES
            off_v[r, pl.ds(cb, _LANES)] = off
            val_v[r, pl.ds(cb, _LANES)] = val
        copies = [
            pltpu.async_copy(val_v.at[j], buf_hbm.at[off_v.at[j]], sem)
            for j in range(n_rows)
        ]
        for c in copies:
            c.wait()

    return sc_scatter


def kernel(obj_sem_cls_pred, obj_labels, cur_step, total_steps):
    B, N, C = obj_sem_cls_pred.shape
    zeros = pl.pallas_call(
        _memset_body,
        grid=(B,),
        out_specs=pl.BlockSpec((1, N, C), lambda b: (b, 0, 0)),
        out_shape=jax.ShapeDtypeStruct((B, N, C), obj_sem_cls_pred.dtype),
    )()
    labf = obj_labels.astype(jnp.int32).reshape(-1)
    buf = jax.new_ref(zeros.reshape(-1))
    _make_sc_scatter(B * N, C)(labf, buf)
    return buf[...].reshape(B, N, C)


# speedup vs baseline: 5.1619x; 5.1619x over previous
import functools

import jax
import jax.numpy as jnp
from jax import lax
from jax.experimental import pallas as pl
from jax.experimental.pallas import tpu as pltpu
from jax.experimental.pallas import tpu_sc as plsc

_NUM_CORES = 2
_NUM_SUBCORES = 16
_LANES = 16
_NW = _NUM_CORES * _NUM_SUBCORES
_ROWS = 32


@functools.cache
def _make_sc_onehot(BN, C):
    per_w = BN // _NW
    n_slabs = per_w // _ROWS
    mesh = plsc.VectorSubcoreMesh(core_axis_name="c", subcore_axis_name="s")

    @functools.partial(
        pl.kernel,
        out_type=jax.ShapeDtypeStruct((BN, C), jnp.float32),
        mesh=mesh,
        compiler_params=pltpu.CompilerParams(
            use_tc_tiling_on_sc=True, needs_layout_passes=False
        ),
        scratch_types=[
            pltpu.VMEM((per_w,), jnp.int32),
            pltpu.VMEM((2, _ROWS, C), jnp.float32),
            pltpu.SemaphoreType.DMA,
            pltpu.SemaphoreType.DMA,
        ],
    )
    def sc_onehot(lab_hbm, zslab_hbm, out_hbm, lab_v, buf, sem0, sem1):
        wid = lax.axis_index("s") * _NUM_CORES + lax.axis_index("c")
        base = wid * per_w
        pltpu.sync_copy(lab_hbm.at[pl.ds(base, per_w)], lab_v)
        pltpu.sync_copy(zslab_hbm, buf.at[0])
        pltpu.sync_copy(zslab_hbm, buf.at[1])
        iota = lax.iota(jnp.int32, _LANES)
        sems = (sem0, sem1)

        def plant(m, s, value):
            for j in range(_ROWS // _LANES):
                lab = lab_v[pl.ds(s * _ROWS + j * _LANES, _LANES)]
                valid = lab >= 0
                col = jnp.where(valid, lab, 0)
                row = iota + j * _LANES
                val = jnp.where(valid, jnp.float32(value), jnp.float32(0.0))
                plsc.store_scatter(buf.at[m], [row, col], val)

        copies = [None, None]
        for s in range(n_slabs):
            m = s % 2
            if copies[m] is not None:
                copies[m].wait()
                plant(m, s - 2, 0.0)
            plant(m, s, 1.0)
            copies[m] = pltpu.async_copy(
                buf.at[m], out_hbm.at[pl.ds(base + s * _ROWS, _ROWS)], sems[m]
            )
        copies[0].wait()
        copies[1].wait()

    return sc_onehot


def kernel(obj_sem_cls_pred, obj_labels, cur_step, total_steps):
    B, N, C = obj_sem_cls_pred.shape
    BN = B * N
    labf = obj_labels.astype(jnp.int32).reshape(BN)
    zslab = jnp.zeros((_ROWS, C), jnp.float32)
    out2 = _make_sc_onehot(BN, C)(labf, zslab)
    return out2.reshape(B, N, C)
